# baseline (speedup 1.0000x reference)
import functools

import jax
import jax.numpy as jnp
from jax import lax
from jax.experimental import pallas as pl
from jax.experimental.pallas import tpu as pltpu
from jax.experimental.pallas import tpu_sc as plsc


def _make_sc_gather(B, n_fields, n_chunk):
    info = plsc.get_sparse_core_info()
    nc, ns = info.num_cores, info.num_subcores
    nw = nc * ns
    b_per_w = B // nw
    n_chunks = b_per_w // n_chunk
    nf = n_fields
    mesh = plsc.VectorSubcoreMesh(core_axis_name="c", subcore_axis_name="s")

    @functools.partial(
        pl.kernel,
        mesh=mesh,
        out_type=[
            jax.ShapeDtypeStruct((B, 128), jnp.float32),
            jax.ShapeDtypeStruct((B, 128), jnp.float32),
            jax.ShapeDtypeStruct((B * nf, 128), jnp.float32),
        ],
        scratch_types=[
            [pltpu.VMEM((n_chunk,), jnp.int32)] * 2,
            [pltpu.VMEM((n_chunk,), jnp.int32)] * 2,
            [[pltpu.VMEM((104,), jnp.int32)] * (n_chunk * nf // 104)] * 2,
            pltpu.VMEM((2, n_chunk, 128), jnp.float32),
            pltpu.VMEM((2, n_chunk, 128), jnp.float32),
            pltpu.VMEM((2, n_chunk * nf, 128), jnp.float32),
            pltpu.SemaphoreType.DMA,
            pltpu.SemaphoreType.DMA,
            pltpu.SemaphoreType.DMA,
            pltpu.SemaphoreType.DMA,
        ],
    )
    def gather_kernel(idx_u_hbm, idx_i_hbm, idx_f_hbm,
                      user_hbm, item_hbm, feat_hbm,
                      out_u, out_i, out_f,
                      idx_u_v, idx_i_v, idx_f_v,
                      u_buf, i_buf, f_buf,
                      sem_idx, sem_u, sem_i, sem_f):

        wid = lax.axis_index("s") * nc + lax.axis_index("c")
        base = wid * b_per_w
        nsub = n_chunk * nf // 104

        def load_idx(c, s):
            r0 = base + c * n_chunk
            pltpu.async_copy(idx_u_hbm.at[pl.ds(r0, n_chunk)],
                             idx_u_v[s], sem_idx)
            pltpu.async_copy(idx_i_hbm.at[pl.ds(r0, n_chunk)],
                             idx_i_v[s], sem_idx)
            for k in range(nsub):
                pltpu.async_copy(
                    idx_f_hbm.at[pl.ds(r0 * nf + k * 104, 104)],
                    idx_f_v[s][k], sem_idx)

        def wait_idx(s):
            pltpu.make_async_copy(idx_u_hbm.at[pl.ds(0, n_chunk)],
                                  idx_u_v[s], sem_idx).wait()
            pltpu.make_async_copy(idx_i_hbm.at[pl.ds(0, n_chunk)],
                                  idx_i_v[s], sem_idx).wait()
            for k in range(nsub):
                pltpu.make_async_copy(idx_f_hbm.at[pl.ds(0, 104)],
                                      idx_f_v[s][k], sem_idx).wait()

        def gather(s):
            pltpu.async_copy(user_hbm.at[idx_u_v[s]], u_buf.at[s], sem_u)
            pltpu.async_copy(item_hbm.at[idx_i_v[s]], i_buf.at[s], sem_i)
            for k in range(nsub):
                pltpu.async_copy(
                    feat_hbm.at[idx_f_v[s][k]],
                    f_buf.at[s, pl.ds(k * 104, 104)], sem_f)

        def wait_gather(s):
            pltpu.make_async_copy(user_hbm.at[idx_u_v[s]],
                                  u_buf.at[s], sem_u).wait()
            pltpu.make_async_copy(item_hbm.at[idx_i_v[s]],
                                  i_buf.at[s], sem_i).wait()
            for k in range(nsub):
                pltpu.make_async_copy(
                    feat_hbm.at[idx_f_v[s][k]],
                    f_buf.at[s, pl.ds(k * 104, 104)], sem_f).wait()

        def writeback(c, slot):
            r0 = base + c * n_chunk
            pltpu.async_copy(u_buf.at[slot], out_u.at[pl.ds(r0, n_chunk)],
                             sem_u)
            pltpu.async_copy(i_buf.at[slot], out_i.at[pl.ds(r0, n_chunk)],
                             sem_i)
            pltpu.async_copy(f_buf.at[slot],
                             out_f.at[pl.ds(r0 * nf, n_chunk * nf)], sem_f)

        def wait_writeback(c, slot):
            r0 = base + c * n_chunk
            pltpu.make_async_copy(u_buf.at[slot],
                                  out_u.at[pl.ds(r0, n_chunk)], sem_u).wait()
            pltpu.make_async_copy(i_buf.at[slot],
                                  out_i.at[pl.ds(r0, n_chunk)], sem_i).wait()
            pltpu.make_async_copy(f_buf.at[slot],
                                  out_f.at[pl.ds(r0 * nf, n_chunk * nf)],
                                  sem_f).wait()

        load_idx(0, 0)
        wait_idx(0)
        gather(0)
        load_idx(1, 1)

        def chunk_ops(c, s):
            ns = 1 - s
            wait_gather(s)
            wait_idx(ns)
            gather(ns)

            @pl.when(c > 0)
            def _():
                wait_writeback(c - 1, ns)

            writeback(c, s)

            @pl.when(c + 2 < n_chunks)
            def _():
                load_idx(c + 2, s)

        def step(c, _):
            @pl.when(lax.rem(c, 2) == 0)
            def _():
                chunk_ops(c, 0)

            @pl.when(lax.rem(c, 2) == 1)
            def _():
                chunk_ops(c, 1)
            return _

        lax.fori_loop(0, n_chunks - 1, step, None, unroll=False)

        last = n_chunks - 1
        lslot = (n_chunks - 1) % 2
        wait_gather(lslot)
        wait_writeback(last - 1, 1 - lslot)
        writeback(last, lslot)
        wait_writeback(last, lslot)

    return gather_kernel


def _make_fm_body(n_fields, vec, sub_width):
    nf2 = n_fields + 2

    def body(ru_ref, ri_ref, rf_ref, sub_ref, k_ref, w_ref, b_ref, o_ref):
        kk = k_ref[...]
        wv = w_ref[...]
        pm = jnp.concatenate([kk, wv], axis=1)
        s2 = jnp.sum(kk * kk, axis=1, keepdims=True)
        bt = ru_ref.shape[0]
        scol = lax.broadcasted_iota(jnp.int32, (1, sub_width), 1) // vec

        def extract(r128, s):
            m = (scol == s).astype(jnp.float32)
            x = r128 * m
            e = x[:, 0:vec]
            for t in range(1, sub_width // vec):
                e = e + x[:, t * vec:(t + 1) * vec]
            return e

        p = None
        q = None
        for f in range(nf2):
            if f == 0:
                r128 = ru_ref[...]
            elif f == 1:
                r128 = ri_ref[...]
            else:
                r128 = rf_ref[:, (f - 2) * sub_width:(f - 1) * sub_width]
            e = extract(r128, sub_ref[:, f:f + 1])
            rows = slice(f * vec, (f + 1) * vec)
            dp = jnp.dot(e, pm[rows], preferred_element_type=jnp.float32)
            dq = jnp.dot(e * e, s2[rows], preferred_element_type=jnp.float32)
            p = dp if p is None else p + dp
            q = dq if q is None else q + dq

        pk = p[:, :-1]
        lin = p[:, -1:]
        cross = 0.5 * (jnp.sum(pk * pk, axis=1, keepdims=True) - q)
        o_ref[...] = jax.nn.sigmoid(lin + b_ref[...] + cross)

    return body


def kernel(inputs, user_table, item_table, feat_tables, w, b, k_mat):
    B = inputs.shape[0]
    n_fields = feat_tables.shape[0]
    vocab = feat_tables.shape[1]
    vec = feat_tables.shape[2]
    total_dim = (2 + n_fields) * vec
    rpp = 128 // vec

    ii = inputs.astype(jnp.int32)
    offs = (jnp.arange(n_fields, dtype=jnp.int32) * vocab)[None, :]
    flat_f = ii[:, 2:] + offs
    idx_u = ii[:, 0] // rpp
    idx_i = ii[:, 1] // rpp
    idx_f = (flat_f // rpp).reshape(-1)
    sub = jnp.concatenate(
        [ii[:, 0:1] % rpp, ii[:, 1:2] % rpp, flat_f % rpp], axis=1)

    u128 = user_table.reshape(user_table.shape[0] // rpp, 128)
    i128 = item_table.reshape(item_table.shape[0] // rpp, 128)
    f128 = feat_tables.reshape(n_fields * vocab // rpp, 128)

    gather = _make_sc_gather(B, n_fields, n_chunk=16)
    out_u, out_i, out_f = gather(idx_u, idx_i, idx_f, u128, i128, f128)
    rf = out_f.reshape(B, n_fields * 128)

    bt = 512
    b2 = jnp.reshape(b, (1, 1))
    body = _make_fm_body(n_fields, vec, 128)
    y = pl.pallas_call(
        body,
        grid=(B // bt,),
        in_specs=[
            pl.BlockSpec((bt, 128), lambda i: (i, 0)),
            pl.BlockSpec((bt, 128), lambda i: (i, 0)),
            pl.BlockSpec((bt, n_fields * 128), lambda i: (i, 0)),
            pl.BlockSpec((bt, 2 + n_fields), lambda i: (i, 0)),
            pl.BlockSpec((total_dim, k_mat.shape[1]), lambda i: (0, 0)),
            pl.BlockSpec((total_dim, 1), lambda i: (0, 0)),
            pl.BlockSpec((1, 1), lambda i: (0, 0)),
        ],
        out_specs=pl.BlockSpec((bt, 1), lambda i: (i, 0)),
        out_shape=jax.ShapeDtypeStruct((B, 1), jnp.float32),
    )(out_u, out_i, rf, sub, k_mat, w, b2)
    return y
